# Initial kernel scaffold; baseline (speedup 1.0000x reference)
import jax
import jax.numpy as jnp
from jax import lax
from jax.experimental import pallas as pl
from jax.experimental.pallas import tpu as pltpu
from jax.experimental.pallas import tpu_sc as plsc

L = 16


def _make_body(n, d, ninc, nw, chunk):
    rows_per_w = n // nw
    nchunk = rows_per_w // chunk

    def body(y_hbm, grid_hbm, inc_hbm, x_hbm, jac_hbm,
             grid_v, inc_v, y_v, x_v, jac_v):
        cid = lax.axis_index("c")
        sid = lax.axis_index("s")
        wid = sid * 2 + cid
        pltpu.sync_copy(grid_hbm, grid_v)
        pltpu.sync_copy(inc_hbm, inc_v)
        lanes = lax.iota(jnp.int32, L)
        base0 = wid * rows_per_w
        for c in range(nchunk):
            row0 = base0 + c * chunk
            pltpu.sync_copy(y_hbm.at[pl.ds(row0 * d, chunk * d)], y_v)

            def ibody(i, carry):
                jac = None
                for dd in range(d):
                    idx = i * (L * d) + lanes * d + dd
                    yv = plsc.load_gather(y_v, [idx])
                    t = yv * float(ninc)
                    iy = t.astype(jnp.int32)
                    iyc = jnp.minimum(jnp.maximum(iy, 0), ninc - 1)
                    dy = t - iyc.astype(jnp.float32)
                    g = plsc.load_gather(grid_v, [iyc + dd * (ninc + 1)])
                    ig = plsc.load_gather(inc_v, [iyc + dd * ninc])
                    xv = g + ig * dy
                    plsc.store_scatter(x_v, [idx], xv)
                    jf = ig * float(ninc)
                    jac = jf if jac is None else jac * jf
                jac_v[pl.ds(i * L, L)] = jac
                return carry

            lax.fori_loop(0, chunk // L, ibody, 0)
            pltpu.sync_copy(x_v, x_hbm.at[pl.ds(row0 * d, chunk * d)])
            pltpu.sync_copy(jac_v, jac_hbm.at[pl.ds(row0, chunk)])

    return body


def kernel(y, grid, inc):
    n, d = y.shape
    ninc = inc.shape[1]
    nw = 32
    rows_per_w = n // nw
    chunk = min(2048, rows_per_w)
    mesh = plsc.VectorSubcoreMesh(core_axis_name="c", subcore_axis_name="s")
    k = pl.kernel(
        _make_body(n, d, ninc, nw, chunk),
        out_type=[
            jax.ShapeDtypeStruct((n * d,), jnp.float32),
            jax.ShapeDtypeStruct((n,), jnp.float32),
        ],
        mesh=mesh,
        scratch_types=[
            pltpu.VMEM((d * (ninc + 1),), jnp.float32),
            pltpu.VMEM((d * ninc,), jnp.float32),
            pltpu.VMEM((chunk * d,), jnp.float32),
            pltpu.VMEM((chunk * d,), jnp.float32),
            pltpu.VMEM((chunk,), jnp.float32),
        ],
    )
    xf, jac = k(y.reshape(-1), grid.reshape(-1), inc.reshape(-1))
    return xf.reshape(n, d), jac


# speedup vs baseline: 145.0119x; 145.0119x over previous
import jax
import jax.numpy as jnp
from jax import lax
from jax.experimental import pallas as pl
from jax.experimental.pallas import tpu as pltpu
from jax.experimental.pallas import tpu_sc as plsc

L = 16


def _make_body(n, d, ninc, nw, chunk):
    rows_per_w = n // nw
    nchunk = rows_per_w // chunk

    def body(y_hbm, grid_hbm, inc_hbm, x_hbm, jac_hbm,
             grid_v, inc_v, y_v, x_v, jac_v):
        cid = lax.axis_index("c")
        sid = lax.axis_index("s")
        wid = sid * 2 + cid
        pltpu.sync_copy(grid_hbm, grid_v)
        pltpu.sync_copy(inc_hbm, inc_v)
        lanes = lax.iota(jnp.int32, L)
        base0 = wid * rows_per_w
        for c in range(nchunk):
            row0 = base0 + c * chunk
            pltpu.sync_copy(y_hbm.at[pl.ds(row0 * d, chunk * d)], y_v)

            def ibody(i, carry):
                jac = None
                for dd in range(d):
                    idx = i * (L * d) + lanes * d + dd
                    yv = plsc.load_gather(y_v, [idx])
                    t = yv * float(ninc)
                    iy = t.astype(jnp.int32)
                    iyc = jnp.minimum(jnp.maximum(iy, 0), ninc - 1)
                    dy = t - iyc.astype(jnp.float32)
                    g = plsc.load_gather(grid_v, [iyc + dd * (ninc + 1)])
                    ig = plsc.load_gather(inc_v, [iyc + dd * ninc])
                    xv = g + ig * dy
                    plsc.store_scatter(x_v, [idx], xv)
                    jf = ig * float(ninc)
                    jac = jf if jac is None else jac * jf
                jac_v[pl.ds(i * L, L)] = jac
                return carry

            lax.fori_loop(0, chunk // L, ibody, 0)
            pltpu.sync_copy(x_v, x_hbm.at[pl.ds(row0 * d, chunk * d)])
            pltpu.sync_copy(jac_v, jac_hbm.at[pl.ds(row0, chunk)])

    return body


def kernel(y, grid, inc):
    n, d = y.shape
    ninc = inc.shape[1]
    nw = 32
    rows_per_w = n // nw
    chunk = min(2048, rows_per_w)
    mesh = plsc.VectorSubcoreMesh(
        core_axis_name="c", subcore_axis_name="s", num_cores=2, num_subcores=16
    )
    k = pl.kernel(
        _make_body(n, d, ninc, nw, chunk),
        out_type=[
            jax.ShapeDtypeStruct((n * d,), jnp.float32),
            jax.ShapeDtypeStruct((n,), jnp.float32),
        ],
        mesh=mesh,
        compiler_params=pltpu.CompilerParams(needs_layout_passes=False),
        scratch_types=[
            pltpu.VMEM((d * (ninc + 1),), jnp.float32),
            pltpu.VMEM((d * ninc,), jnp.float32),
            pltpu.VMEM((chunk * d,), jnp.float32),
            pltpu.VMEM((chunk * d,), jnp.float32),
            pltpu.VMEM((chunk,), jnp.float32),
        ],
    )
    xf, jac = k(y.reshape(-1), grid.reshape(-1), inc.reshape(-1))
    return xf.reshape(n, d), jac


# speedup vs baseline: 179.3251x; 1.2366x over previous
import jax
import jax.numpy as jnp
from jax import lax
from jax.experimental import pallas as pl
from jax.experimental.pallas import tpu as pltpu
from jax.experimental.pallas import tpu_sc as plsc

L = 16


def _make_body(n, d, ninc, nw, chunk):
    rows_per_w = n // nw
    nchunk = rows_per_w // chunk

    def body(y_hbm, grid_hbm, inc_hbm, x_hbm, jac_hbm,
             grid_v, inc_v, y_v, x_v, jac_v):
        cid = lax.axis_index("c")
        sid = lax.axis_index("s")
        wid = sid * 2 + cid
        pltpu.sync_copy(grid_hbm, grid_v)
        pltpu.sync_copy(inc_hbm, inc_v)
        lanes = lax.iota(jnp.int32, L)
        base0 = wid * rows_per_w

        def cbody(c, carry):
            row0 = pl.multiple_of(base0 + c * chunk, chunk)
            pltpu.sync_copy(y_hbm.at[pl.ds(row0 * d, chunk * d)], y_v)

            @plsc.parallel_loop(0, chunk // L, unroll=2)
            def ibody(i):
                jac = None
                for dd in range(d):
                    idx = i * (L * d) + lanes * d + dd
                    yv = plsc.load_gather(y_v, [idx])
                    t = yv * float(ninc)
                    iy = t.astype(jnp.int32)
                    iyc = jnp.minimum(jnp.maximum(iy, 0), ninc - 1)
                    dy = t - iyc.astype(jnp.float32)
                    g = plsc.load_gather(grid_v, [iyc + dd * (ninc + 1)])
                    ig = plsc.load_gather(inc_v, [iyc + dd * ninc])
                    xv = g + ig * dy
                    plsc.store_scatter(x_v, [idx], xv)
                    jf = ig * float(ninc)
                    jac = jf if jac is None else jac * jf
                jac_v[pl.ds(i * L, L)] = jac
            pltpu.sync_copy(x_v, x_hbm.at[pl.ds(row0 * d, chunk * d)])
            pltpu.sync_copy(jac_v, jac_hbm.at[pl.ds(row0, chunk)])
            return carry

        lax.fori_loop(0, nchunk, cbody, 0)

    return body


def kernel(y, grid, inc):
    n, d = y.shape
    ninc = inc.shape[1]
    nw = 32
    rows_per_w = n // nw
    chunk = min(2048, rows_per_w)
    mesh = plsc.VectorSubcoreMesh(
        core_axis_name="c", subcore_axis_name="s", num_cores=2, num_subcores=16
    )
    k = pl.kernel(
        _make_body(n, d, ninc, nw, chunk),
        out_type=[
            jax.ShapeDtypeStruct((n * d,), jnp.float32),
            jax.ShapeDtypeStruct((n,), jnp.float32),
        ],
        mesh=mesh,
        compiler_params=pltpu.CompilerParams(needs_layout_passes=False),
        scratch_types=[
            pltpu.VMEM((d * (ninc + 1),), jnp.float32),
            pltpu.VMEM((d * ninc,), jnp.float32),
            pltpu.VMEM((chunk * d,), jnp.float32),
            pltpu.VMEM((chunk * d,), jnp.float32),
            pltpu.VMEM((chunk,), jnp.float32),
        ],
    )
    xf, jac = k(y.reshape(-1), grid.reshape(-1), inc.reshape(-1))
    return xf.reshape(n, d), jac


# speedup vs baseline: 181.7269x; 1.0134x over previous
import jax
import jax.numpy as jnp
from jax import lax
from jax.experimental import pallas as pl
from jax.experimental.pallas import tpu as pltpu
from jax.experimental.pallas import tpu_sc as plsc

L = 16


def _make_body(n, d, ninc, nw, chunk):
    rows_per_w = n // nw
    nchunk = rows_per_w // chunk

    def body(y_hbm, grid_hbm, inc_hbm, x_hbm, jac_hbm,
             grid_v, inc_v, y_v, x_v, jac_v):
        cid = lax.axis_index("c")
        sid = lax.axis_index("s")
        wid = sid * 2 + cid
        pltpu.sync_copy(grid_hbm, grid_v)
        pltpu.sync_copy(inc_hbm, inc_v)
        lanes = lax.iota(jnp.int32, L)
        base0 = wid * rows_per_w

        def cbody(c, carry):
            row0 = pl.multiple_of(base0 + c * chunk, chunk)
            pltpu.sync_copy(y_hbm.at[pl.ds(row0, chunk), :], y_v)

            @plsc.parallel_loop(0, chunk // L, unroll=2)
            def ibody(i):
                rows = i * L + lanes
                jac = None
                for dd in range(d):
                    dvec = jnp.full((L,), dd, dtype=jnp.int32)
                    yv = plsc.load_gather(y_v, [rows, dvec])
                    t = yv * float(ninc)
                    iy = t.astype(jnp.int32)
                    iyc = jnp.minimum(jnp.maximum(iy, 0), ninc - 1)
                    dy = t - iyc.astype(jnp.float32)
                    g = plsc.load_gather(grid_v, [dvec, iyc])
                    ig = plsc.load_gather(inc_v, [dvec, iyc])
                    xv = g + ig * dy
                    plsc.store_scatter(x_v, [rows, dvec], xv)
                    jf = ig * float(ninc)
                    jac = jf if jac is None else jac * jf
                jac_v[pl.ds(i * L, L)] = jac

            pltpu.sync_copy(x_v, x_hbm.at[pl.ds(row0, chunk), :])
            pltpu.sync_copy(jac_v, jac_hbm.at[pl.ds(row0, chunk)])
            return carry

        lax.fori_loop(0, nchunk, cbody, 0)

    return body


def kernel(y, grid, inc):
    n, d = y.shape
    ninc = inc.shape[1]
    nw = 32
    rows_per_w = n // nw
    chunk = min(2048, rows_per_w)
    mesh = plsc.VectorSubcoreMesh(
        core_axis_name="c", subcore_axis_name="s", num_cores=2, num_subcores=16
    )
    k = pl.kernel(
        _make_body(n, d, ninc, nw, chunk),
        out_type=[
            jax.ShapeDtypeStruct((n, d), jnp.float32),
            jax.ShapeDtypeStruct((n,), jnp.float32),
        ],
        mesh=mesh,
        compiler_params=pltpu.CompilerParams(
            needs_layout_passes=False, use_tc_tiling_on_sc=False
        ),
        scratch_types=[
            pltpu.VMEM((d, ninc + 1), jnp.float32),
            pltpu.VMEM((d, ninc), jnp.float32),
            pltpu.VMEM((chunk, d), jnp.float32),
            pltpu.VMEM((chunk, d), jnp.float32),
            pltpu.VMEM((chunk,), jnp.float32),
        ],
    )
    x, jac = k(y, grid, inc)
    return x, jac


# speedup vs baseline: 1583.8292x; 8.7154x over previous
import jax
import jax.numpy as jnp
from jax import lax
from jax.experimental import pallas as pl
from jax.experimental.pallas import tpu as pltpu
from jax.experimental.pallas import tpu_sc as plsc

L = 16
TL = 128


def _make_body(n, d, ninc, nw, chunk):
    rows_per_w = n // nw
    nchunk = rows_per_w // chunk
    tiles_per_chunk = chunk // TL
    groups_per_tile = TL // L

    def body(y_hbm, grid_hbm, inc_hbm, x_hbm, jac_hbm,
             grid_v, inc_v, y_v, x_v, jac_v):
        cid = lax.axis_index("c")
        sid = lax.axis_index("s")
        wid = sid * 2 + cid
        pltpu.sync_copy(grid_hbm, grid_v)
        pltpu.sync_copy(inc_hbm, inc_v)
        base0 = wid * (rows_per_w // TL)

        def cbody(c, carry):
            t0 = pl.multiple_of(base0 + c * tiles_per_chunk, tiles_per_chunk)
            pltpu.sync_copy(y_hbm.at[pl.ds(t0, tiles_per_chunk)], y_v)

            @plsc.parallel_loop(0, chunk // L, unroll=2)
            def ibody(i):
                t = i // groups_per_tile
                l0 = (i % groups_per_tile) * L
                jac = None
                for dd in range(d):
                    dvec = jnp.full((L,), dd, dtype=jnp.int32)
                    yv = y_v[t, dd, pl.ds(l0, L)]
                    tt = yv * float(ninc)
                    iy = tt.astype(jnp.int32)
                    iyc = jnp.minimum(jnp.maximum(iy, 0), ninc - 1)
                    dy = tt - iyc.astype(jnp.float32)
                    g = plsc.load_gather(grid_v, [dvec, iyc])
                    ig = plsc.load_gather(inc_v, [dvec, iyc])
                    x_v[t, dd, pl.ds(l0, L)] = g + ig * dy
                    jf = ig * float(ninc)
                    jac = jf if jac is None else jac * jf
                jac_v[pl.ds(i * L, L)] = jac

            pltpu.sync_copy(x_v, x_hbm.at[pl.ds(t0, tiles_per_chunk)])
            pltpu.sync_copy(
                jac_v, jac_hbm.at[pl.ds(pl.multiple_of(t0 * TL, chunk), chunk)]
            )
            return carry

        lax.fori_loop(0, nchunk, cbody, 0)

    return body


def kernel(y, grid, inc):
    n, d = y.shape
    ninc = inc.shape[1]
    nw = 32
    rows_per_w = n // nw
    chunk = min(2048, rows_per_w)
    nt = n // TL
    y3 = y.T.reshape(d, nt, TL).transpose(1, 0, 2)
    mesh = plsc.VectorSubcoreMesh(
        core_axis_name="c", subcore_axis_name="s", num_cores=2, num_subcores=16
    )
    k = pl.kernel(
        _make_body(n, d, ninc, nw, chunk),
        out_type=[
            jax.ShapeDtypeStruct((nt, d, TL), jnp.float32),
            jax.ShapeDtypeStruct((n,), jnp.float32),
        ],
        mesh=mesh,
        compiler_params=pltpu.CompilerParams(
            needs_layout_passes=False, use_tc_tiling_on_sc=False
        ),
        scratch_types=[
            pltpu.VMEM((d, ninc + 1), jnp.float32),
            pltpu.VMEM((d, ninc), jnp.float32),
            pltpu.VMEM((chunk // TL, d, TL), jnp.float32),
            pltpu.VMEM((chunk // TL, d, TL), jnp.float32),
            pltpu.VMEM((chunk,), jnp.float32),
        ],
    )
    x3, jac = k(y3, grid, inc)
    x = x3.transpose(1, 0, 2).reshape(d, n).T
    return x, jac


# speedup vs baseline: 2177.9190x; 1.3751x over previous
import jax
import jax.numpy as jnp
from jax import lax
from jax.experimental import pallas as pl
from jax.experimental.pallas import tpu as pltpu
from jax.experimental.pallas import tpu_sc as plsc

L = 16
TL = 128


def _make_body(n, d, ninc, nw, chunk):
    rows_per_w = n // nw
    nchunk = rows_per_w // chunk
    tpc = chunk // TL
    gpt_shift = 3
    gpt_mask = (TL // L) - 1

    def body(y_hbm, grid_hbm, inc_hbm, x_hbm, jac_hbm, grid_v, inc_v,
             y_b, x_b, jac_b, sem_in, sem_x, sem_jac):
        cid = lax.axis_index("c")
        sid = lax.axis_index("s")
        wid = sid * 2 + cid
        pltpu.sync_copy(grid_hbm, grid_v)
        pltpu.sync_copy(inc_hbm, inc_v)
        base0 = wid * (rows_per_w // TL)

        def tile0(c):
            return pl.multiple_of(base0 + c * tpc, tpc)

        def start_in(c, b):
            pltpu.async_copy(y_hbm.at[pl.ds(tile0(c), tpc)], y_b[b], sem_in[b])

        def wait_in(c, b):
            pltpu.make_async_copy(
                y_hbm.at[pl.ds(tile0(c), tpc)], y_b[b], sem_in[b]
            ).wait()

        def start_out(c, b):
            t0 = tile0(c)
            pltpu.async_copy(x_b[b], x_hbm.at[pl.ds(t0, tpc)], sem_x[b])
            r0 = pl.multiple_of(t0 * TL, chunk)
            pltpu.async_copy(jac_b[b], jac_hbm.at[pl.ds(r0, chunk)], sem_jac[b])

        def wait_out(c, b):
            t0 = tile0(c)
            pltpu.make_async_copy(
                x_b[b], x_hbm.at[pl.ds(t0, tpc)], sem_x[b]
            ).wait()
            r0 = pl.multiple_of(t0 * TL, chunk)
            pltpu.make_async_copy(
                jac_b[b], jac_hbm.at[pl.ds(r0, chunk)], sem_jac[b]
            ).wait()

        def compute(b):
            y_v, x_v, jac_v = y_b[b], x_b[b], jac_b[b]

            @plsc.parallel_loop(0, chunk // L, unroll=2)
            def ibody(i):
                t = i >> gpt_shift
                l0 = (i & gpt_mask) * L
                jac = None
                for dd in range(d):
                    dvec = jnp.full((L,), dd, dtype=jnp.int32)
                    yv = y_v[t, dd, pl.ds(l0, L)]
                    tt = yv * float(ninc)
                    iy = tt.astype(jnp.int32)
                    iyc = jnp.minimum(jnp.maximum(iy, 0), ninc - 1)
                    dy = tt - iyc.astype(jnp.float32)
                    g = plsc.load_gather(grid_v, [dvec, iyc])
                    ig = plsc.load_gather(inc_v, [dvec, iyc])
                    x_v[t, dd, pl.ds(l0, L)] = g + ig * dy
                    jf = ig * float(ninc)
                    jac = jf if jac is None else jac * jf
                jac_v[pl.ds(i * L, L)] = jac

        start_in(0, 0)

        def cbody(h, carry):
            for b in range(2):
                c = h * 2 + b
                wait_in(c, b)

                @pl.when(c + 1 < nchunk)
                def _():
                    start_in(c + 1, 1 - b)

                @pl.when(c >= 2)
                def _():
                    wait_out(c - 2, b)

                compute(b)
                start_out(c, b)
            return carry

        lax.fori_loop(0, nchunk // 2, cbody, 0)
        wait_out(nchunk - 2, 0)
        wait_out(nchunk - 1, 1)

    return body


def kernel(y, grid, inc):
    n, d = y.shape
    ninc = inc.shape[1]
    nw = 32
    rows_per_w = n // nw
    chunk = min(2048, rows_per_w)
    nt = n // TL
    y3 = y.T.reshape(d, nt, TL).transpose(1, 0, 2)
    mesh = plsc.VectorSubcoreMesh(
        core_axis_name="c", subcore_axis_name="s", num_cores=2, num_subcores=16
    )
    k = pl.kernel(
        _make_body(n, d, ninc, nw, chunk),
        out_type=[
            jax.ShapeDtypeStruct((nt, d, TL), jnp.float32),
            jax.ShapeDtypeStruct((n,), jnp.float32),
        ],
        mesh=mesh,
        compiler_params=pltpu.CompilerParams(
            needs_layout_passes=False, use_tc_tiling_on_sc=False
        ),
        scratch_types=[
            pltpu.VMEM((d, ninc + 1), jnp.float32),
            pltpu.VMEM((d, ninc), jnp.float32),
            [pltpu.VMEM((chunk // TL, d, TL), jnp.float32) for _ in range(2)],
            [pltpu.VMEM((chunk // TL, d, TL), jnp.float32) for _ in range(2)],
            [pltpu.VMEM((chunk,), jnp.float32) for _ in range(2)],
            [pltpu.SemaphoreType.DMA for _ in range(2)],
            [pltpu.SemaphoreType.DMA for _ in range(2)],
            [pltpu.SemaphoreType.DMA for _ in range(2)],
        ],
    )
    x3, jac = k(y3, grid, inc)
    x = x3.transpose(1, 0, 2).reshape(d, n).T
    return x, jac


# speedup vs baseline: 2182.4904x; 1.0021x over previous
import jax
import jax.numpy as jnp
from jax import lax
from jax.experimental import pallas as pl
from jax.experimental.pallas import tpu as pltpu
from jax.experimental.pallas import tpu_sc as plsc

L = 16
TL = 128


def _make_body(n, d, ninc, nw, chunk):
    rows_per_w = n // nw
    nchunk = rows_per_w // chunk
    tpc = chunk // TL
    gpt_shift = 3
    gpt_mask = (TL // L) - 1

    def body(y_hbm, grid_hbm, inc_hbm, x_hbm, jac_hbm, grid_v, inc_v,
             y_b, x_b, jac_b, sem_in, sem_x, sem_jac):
        cid = lax.axis_index("c")
        sid = lax.axis_index("s")
        wid = sid * 2 + cid
        pltpu.sync_copy(grid_hbm, grid_v)
        pltpu.sync_copy(inc_hbm, inc_v)
        base0 = wid * (rows_per_w // TL)

        def tile0(c):
            return pl.multiple_of(base0 + c * tpc, tpc)

        def start_in(c, b):
            pltpu.async_copy(y_hbm.at[pl.ds(tile0(c), tpc)], y_b[b], sem_in[b])

        def wait_in(c, b):
            pltpu.make_async_copy(
                y_hbm.at[pl.ds(tile0(c), tpc)], y_b[b], sem_in[b]
            ).wait()

        def start_out(c, b):
            t0 = tile0(c)
            pltpu.async_copy(x_b[b], x_hbm.at[pl.ds(t0, tpc)], sem_x[b])
            r0 = pl.multiple_of(t0 * TL, chunk)
            pltpu.async_copy(jac_b[b], jac_hbm.at[pl.ds(r0, chunk)], sem_jac[b])

        def wait_out(c, b):
            t0 = tile0(c)
            pltpu.make_async_copy(
                x_b[b], x_hbm.at[pl.ds(t0, tpc)], sem_x[b]
            ).wait()
            r0 = pl.multiple_of(t0 * TL, chunk)
            pltpu.make_async_copy(
                jac_b[b], jac_hbm.at[pl.ds(r0, chunk)], sem_jac[b]
            ).wait()

        def compute(b):
            y_v, x_v, jac_v = y_b[b], x_b[b], jac_b[b]

            @plsc.parallel_loop(0, chunk // L, unroll=4)
            def ibody(i):
                t = i >> gpt_shift
                l0 = (i & gpt_mask) * L
                jac = None
                for dd in range(d):
                    yv = y_v[t, dd, pl.ds(l0, L)]
                    tt = yv * float(ninc)
                    iy = tt.astype(jnp.int32)
                    iyc = jnp.minimum(jnp.maximum(iy, 0), ninc - 1)
                    dy = tt - iyc.astype(jnp.float32)
                    g = plsc.load_gather(grid_v.at[dd], [iyc])
                    ig = plsc.load_gather(inc_v.at[dd], [iyc])
                    x_v[t, dd, pl.ds(l0, L)] = g + ig * dy
                    jf = ig * float(ninc)
                    jac = jf if jac is None else jac * jf
                jac_v[pl.ds(i * L, L)] = jac

        start_in(0, 0)

        def cbody(h, carry):
            for b in range(2):
                c = h * 2 + b
                wait_in(c, b)

                @pl.when(c + 1 < nchunk)
                def _():
                    start_in(c + 1, 1 - b)

                @pl.when(c >= 2)
                def _():
                    wait_out(c - 2, b)

                compute(b)
                start_out(c, b)
            return carry

        lax.fori_loop(0, nchunk // 2, cbody, 0)
        wait_out(nchunk - 2, 0)
        wait_out(nchunk - 1, 1)

    return body


def kernel(y, grid, inc):
    n, d = y.shape
    ninc = inc.shape[1]
    nw = 32
    rows_per_w = n // nw
    chunk = min(2048, rows_per_w)
    nt = n // TL
    y3 = y.T.reshape(d, nt, TL).transpose(1, 0, 2)
    mesh = plsc.VectorSubcoreMesh(
        core_axis_name="c", subcore_axis_name="s", num_cores=2, num_subcores=16
    )
    k = pl.kernel(
        _make_body(n, d, ninc, nw, chunk),
        out_type=[
            jax.ShapeDtypeStruct((nt, d, TL), jnp.float32),
            jax.ShapeDtypeStruct((n,), jnp.float32),
        ],
        mesh=mesh,
        compiler_params=pltpu.CompilerParams(
            needs_layout_passes=False, use_tc_tiling_on_sc=False
        ),
        scratch_types=[
            pltpu.VMEM((d, ninc + 1), jnp.float32),
            pltpu.VMEM((d, ninc), jnp.float32),
            [pltpu.VMEM((chunk // TL, d, TL), jnp.float32) for _ in range(2)],
            [pltpu.VMEM((chunk // TL, d, TL), jnp.float32) for _ in range(2)],
            [pltpu.VMEM((chunk,), jnp.float32) for _ in range(2)],
            [pltpu.SemaphoreType.DMA for _ in range(2)],
            [pltpu.SemaphoreType.DMA for _ in range(2)],
            [pltpu.SemaphoreType.DMA for _ in range(2)],
        ],
    )
    x3, jac = k(y3, grid, inc)
    x = x3.transpose(1, 0, 2).reshape(d, n).T
    return x, jac
